# Initial kernel scaffold; baseline (speedup 1.0000x reference)
import jax
import jax.numpy as jnp
from jax.experimental import pallas as pl


def kernel(x, edge_index, W1, b1, g1, be1, W2, b2, g2, be2):
    raise NotImplementedError("write your pallas kernel here")


# speedup vs baseline: 6.3643x; 6.3643x over previous
import functools

import jax
import jax.numpy as jnp
from jax import lax
from jax.experimental import pallas as pl
from jax.experimental.pallas import tpu as pltpu
from jax.experimental.pallas import tpu_sc as plsc

N = 10000
E = 320000
NR = 10240
EP = 327680
STRIPE = NR // 16
IN_CH = 128
HID_CH = 256
OUT_CH = 128
EPS = 1e-5


def _make_sc_agg(table_rows, width, n_chunks):
    mesh = plsc.VectorSubcoreMesh(core_axis_name="c", subcore_axis_name="s")
    ib = 8
    n_groups = n_chunks // ib

    def body(table, srcb, dstb, zeros, out, srcv, dstv, rows0, rows1, acc,
             sem0, sem1):
        c = lax.axis_index("c")
        s = lax.axis_index("s")
        pltpu.sync_copy(zeros, acc.at[pl.ds(s * STRIPE, STRIPE)])
        plsc.subcore_barrier()

        def group(g, carry):
            pltpu.sync_copy(srcb.at[c, s, pl.ds(g * ib, ib)], srcv)
            pltpu.sync_copy(dstb.at[c, s, pl.ds(g * ib, ib)], dstv)
            pltpu.async_copy(table.at[srcv.at[0]], rows0, sem0)

            def step(i, carry2):
                j0 = 2 * i
                j1 = j0 + 1
                pltpu.async_copy(table.at[srcv.at[j1]], rows1, sem1)
                pltpu.make_async_copy(table.at[srcv.at[j0]], rows0,
                                      sem0).wait()
                pltpu.sync_copy(rows0, acc.at[dstv.at[j0]], add=True)

                @pl.when(i + 1 < ib // 2)
                def _():
                    pltpu.async_copy(table.at[srcv.at[j0 + 2]], rows0, sem0)

                pltpu.make_async_copy(table.at[srcv.at[j1]], rows1,
                                      sem1).wait()
                pltpu.sync_copy(rows1, acc.at[dstv.at[j1]], add=True)
                return carry2

            lax.fori_loop(0, ib // 2, step, 0)
            return carry

        lax.fori_loop(0, n_groups, group, 0)
        plsc.subcore_barrier()
        pltpu.sync_copy(acc.at[pl.ds(s * STRIPE, STRIPE)],
                        out.at[c, pl.ds(s * STRIPE, STRIPE)])

    return pl.kernel(
        body,
        out_type=jax.ShapeDtypeStruct((2, NR, width), jnp.float32),
        mesh=mesh,
        scratch_types=[
            pltpu.VMEM((ib, 128), jnp.int32),
            pltpu.VMEM((ib, 128), jnp.int32),
            pltpu.VMEM((128, width), jnp.float32),
            pltpu.VMEM((128, width), jnp.float32),
            pltpu.VMEM_SHARED((NR, width), jnp.float32),
            pltpu.SemaphoreType.DMA,
            pltpu.SemaphoreType.DMA,
        ],
    )


_sc_deg = _make_sc_agg(NR, 128, EP // (32 * 128))
_sc_agg1 = _make_sc_agg(2 * NR, IN_CH, EP // (16 * 128))
_sc_agg2 = _make_sc_agg(NR, OUT_CH, EP // (32 * 128))


def _tc_prep(x_ref, w1_ref, degp_ref, y_ref, dinv_ref):
    deg = degp_ref[0, :, 0:1] + degp_ref[1, :, 0:1] + 1.0
    dinv = lax.rsqrt(jnp.maximum(deg, 1.0))
    dinv_ref[...] = dinv
    xw = jnp.dot(x_ref[...], w1_ref[...], preferred_element_type=jnp.float32)
    y = xw * dinv[:N]
    y_ref[0, :N, :] = y[:, :IN_CH]
    y_ref[1, :N, :] = y[:, IN_CH:]


def _tc_mid(agg_ref, y1_ref, dinv_ref, b1_ref, g1_ref, be1_ref, w2_ref,
            y2_ref):
    dinv = dinv_ref[:N]
    hs = []
    for c in range(2):
        sl = slice(c * IN_CH, (c + 1) * IN_CH)
        t = (agg_ref[c, :N, :] + y1_ref[c, :N, :]) * dinv + b1_ref[:, sl]
        m = jnp.mean(t, axis=0, keepdims=True)
        v = jnp.mean(t * t, axis=0, keepdims=True) - m * m
        h = (t - m) * lax.rsqrt(v + EPS) * g1_ref[:, sl] + be1_ref[:, sl]
        hs.append(jnp.maximum(h, 0.0))
    y2 = (jnp.dot(hs[0], w2_ref[:IN_CH, :], preferred_element_type=jnp.float32)
          + jnp.dot(hs[1], w2_ref[IN_CH:, :],
                    preferred_element_type=jnp.float32))
    y2_ref[:N, :] = y2 * dinv


def _tc_fin(aggp_ref, y2_ref, dinv_ref, b2_ref, g2_ref, be2_ref, out_ref):
    dinv = dinv_ref[:N]
    t = ((aggp_ref[0, :N, :] + aggp_ref[1, :N, :] + y2_ref[:N, :]) * dinv
         + b2_ref[...])
    m = jnp.mean(t, axis=0, keepdims=True)
    v = jnp.mean(t * t, axis=0, keepdims=True) - m * m
    out_ref[...] = (t - m) * lax.rsqrt(v + EPS) * g2_ref[...] + be2_ref[...]


def kernel(x, edge_index, W1, b1, g1, be1, W2, b2, g2, be2):
    src = edge_index[0].astype(jnp.int32)
    dst = edge_index[1].astype(jnp.int32)
    pad = EP - E
    src_p = jnp.concatenate([src, jnp.zeros((pad,), jnp.int32)])
    dst_p = jnp.concatenate([dst, jnp.full((pad,), N, jnp.int32)])

    srcb_es = src_p.reshape(2, 16, -1, 128)
    dstb_es = dst_p.reshape(2, 16, -1, 128)
    src_cs = src_p.reshape(1, 16, -1, 128)
    srcb_cs = jnp.concatenate([src_cs, src_cs + NR], axis=0)
    dstb_cs = jnp.broadcast_to(dst_p.reshape(1, 16, -1, 128),
                               (2, 16, EP // (16 * 128), 128))

    ones_t = jnp.ones((NR, 128), jnp.float32)
    z128 = jnp.zeros((STRIPE, 128), jnp.float32)

    degp = _sc_deg(ones_t, srcb_es, dstb_es, z128)

    y1_tab, dinv = pl.pallas_call(
        _tc_prep,
        out_shape=(jax.ShapeDtypeStruct((2, NR, IN_CH), jnp.float32),
                   jax.ShapeDtypeStruct((NR, 1), jnp.float32)),
    )(x, W1, degp)

    agg1 = _sc_agg1(y1_tab.reshape(2 * NR, IN_CH), srcb_cs, dstb_cs, z128)

    y2_tab = pl.pallas_call(
        _tc_mid,
        out_shape=jax.ShapeDtypeStruct((NR, OUT_CH), jnp.float32),
    )(agg1, y1_tab, dinv, b1.reshape(1, -1), g1.reshape(1, -1),
      be1.reshape(1, -1), W2)

    agg2 = _sc_agg2(y2_tab, srcb_es, dstb_es, z128)

    out = pl.pallas_call(
        _tc_fin,
        out_shape=jax.ShapeDtypeStruct((N, OUT_CH), jnp.float32),
    )(agg2, y2_tab, dinv, b2.reshape(1, -1), g2.reshape(1, -1),
      be2.reshape(1, -1))
    return out
